# baseline (speedup 1.0000x reference)
import functools

import jax
import jax.numpy as jnp
from jax import lax
from jax.experimental import pallas as pl
from jax.experimental.pallas import tpu as pltpu

B, C, H, W = 16, 80, 128, 128
K = 100
MAX_OBJS = 128
M = 512

PLANES = B * C
NROW = C * H
BLK = 16
HW = H * W


def _a_kernel(hm_ref, gt_ref, wh_ref, reg_ref, cmax_ref, carg_ref, cg_ref,
              w0_ref, w1_ref, r0_ref, r1_ref, part_ref):
    i = pl.program_id(0)
    x = hm_ref[...]
    g = gt_ref[...]
    s = jnp.clip(jax.nn.sigmoid(x), 1e-4, 1.0 - 1e-4)

    zc = jnp.zeros((BLK, H, 1), jnp.float32)
    left = jnp.concatenate([s[:, :, 1:], zc], axis=2)
    right = jnp.concatenate([zc, s[:, :, :-1]], axis=2)
    hx = jnp.maximum(jnp.maximum(left, right), s)
    zr = jnp.zeros((BLK, 1, W), jnp.float32)
    up = jnp.concatenate([hx[:, 1:, :], zr], axis=1)
    dn = jnp.concatenate([zr, hx[:, :-1, :]], axis=1)
    hmax = jnp.maximum(jnp.maximum(up, dn), hx)
    nm = jnp.where(hmax == s, s, 0.0)

    rmax = jnp.max(nm, axis=2)
    lane = lax.broadcasted_iota(jnp.int32, (BLK, H, W), 2)
    rarg = jnp.min(jnp.where(nm == rmax[:, :, None], lane, W), axis=2)
    onehot = lane == rarg[:, :, None]
    gsel = jnp.sum(jnp.where(onehot, g, 0.0), axis=2)
    cmax_ref[...] = rmax
    carg_ref[...] = rarg.astype(jnp.float32)
    cg_ref[...] = gsel

    def ext(plane):
        return jnp.sum(jnp.where(onehot, plane, 0.0), axis=2)

    w0_ref[...] = ext(wh_ref[0, 0][None])
    w1_ref[...] = ext(wh_ref[0, 1][None])
    r0_ref[...] = ext(reg_ref[0, 0][None])
    r1_ref[...] = ext(reg_ref[0, 1][None])

    pos = (g == 1.0).astype(jnp.float32)
    one_m_g = 1.0 - g
    nw = one_m_g * one_m_g
    nw = nw * nw
    pos_loss = jnp.log(s) * (1.0 - s) * (1.0 - s) * pos
    neg_loss = jnp.log(1.0 - s) * s * s * nw * (1.0 - pos)

    def r(v):
        t = jnp.sum(v, axis=0)
        return jnp.sum(t.reshape(16, 8, 128), axis=0)

    part = jnp.stack([r(pos_loss), r(neg_loss), r(pos)], axis=0)

    @pl.when(i == 0)
    def _():
        part_ref[...] = jnp.zeros_like(part_ref)

    part_ref[...] += part


def _run_a(hm, gt_hm, wh, reg):
    return pl.pallas_call(
        _a_kernel,
        grid=(PLANES // BLK,),
        in_specs=[
            pl.BlockSpec((BLK, H, W), lambda i: (i, 0, 0)),
            pl.BlockSpec((BLK, H, W), lambda i: (i, 0, 0)),
            pl.BlockSpec((1, 2, H, W), lambda i: (i // 5, 0, 0, 0)),
            pl.BlockSpec((1, 2, H, W), lambda i: (i // 5, 0, 0, 0)),
        ],
        out_specs=[
            pl.BlockSpec((BLK, H), lambda i: (i, 0)),
            pl.BlockSpec((BLK, H), lambda i: (i, 0)),
            pl.BlockSpec((BLK, H), lambda i: (i, 0)),
            pl.BlockSpec((BLK, H), lambda i: (i, 0)),
            pl.BlockSpec((BLK, H), lambda i: (i, 0)),
            pl.BlockSpec((BLK, H), lambda i: (i, 0)),
            pl.BlockSpec((BLK, H), lambda i: (i, 0)),
            pl.BlockSpec((3, 8, 128), lambda i: (0, 0, 0)),
        ],
        out_shape=[
            jax.ShapeDtypeStruct((PLANES, H), jnp.float32),
            jax.ShapeDtypeStruct((PLANES, H), jnp.float32),
            jax.ShapeDtypeStruct((PLANES, H), jnp.float32),
            jax.ShapeDtypeStruct((PLANES, H), jnp.float32),
            jax.ShapeDtypeStruct((PLANES, H), jnp.float32),
            jax.ShapeDtypeStruct((PLANES, H), jnp.float32),
            jax.ShapeDtypeStruct((PLANES, H), jnp.float32),
            jax.ShapeDtypeStruct((3, 8, 128), jnp.float32),
        ],
    )(hm.reshape(PLANES, H, W), gt_hm.reshape(PLANES, H, W), wh, reg)


def _s_kernel(cm_ref, t_ref):
    v = cm_ref[...]
    bits = lax.bitcast_convert_type(v, jnp.int32)

    def body(_, lohi):
        lo, hi = lohi
        mid = (lo + hi) // 2
        cnt = jnp.sum((bits >= mid).astype(jnp.int32))
        take = cnt >= K
        return jnp.where(take, mid, lo), jnp.where(take, hi, mid)

    lo, _ = lax.fori_loop(0, 31, body, (jnp.int32(0), jnp.int32(0x3F800001)))
    t_ref[...] = jnp.full((1, 8, 128), lo, jnp.int32)


def _run_s(cmax):
    return pl.pallas_call(
        _s_kernel,
        grid=(B,),
        in_specs=[pl.BlockSpec((1, C, H), lambda i: (i, 0, 0))],
        out_specs=pl.BlockSpec((1, 8, 128), lambda i: (i, 0, 0)),
        out_shape=jax.ShapeDtypeStruct((B, 8, 128), jnp.int32),
    )(cmax.reshape(B, C, H))


def _d_kernel(part_ref, bx1_ref, by1_ref, bx2_ref, by2_ref, valid_ref,
              val_ref, gold_ref, tb_ref,
              pw0_ref, pw1_ref, pr0_ref, pr1_ref,
              tw0_ref, tw1_ref, tr0_ref, tr1_ref, rm_ref, o_ref):
    part = part_ref[...]
    pos_sum = jnp.sum(part[0])
    neg_sum = jnp.sum(part[1])
    num_pos = jnp.sum(part[2])

    tb = tb_ref[...]
    tx1 = tb[0]
    ty1 = tb[1]
    tx2 = tb[2]
    ty2 = tb[3]
    tbi = tb[4]
    a2 = (tx2 - tx1) * (ty2 - ty1)

    bx1 = bx1_ref[...]
    by1 = by1_ref[...]
    bx2 = bx2_ref[...]
    by2 = by2_ref[...]
    valid = valid_ref[...]

    ious = []
    for i in range(B):
        m = (tbi == float(i)).astype(jnp.float32)
        a1 = (bx2[i] - bx1[i]) * (by2[i] - by1[i])
        ltx = jnp.maximum(bx1[i][:, None], tx1[None, :])
        lty = jnp.maximum(by1[i][:, None], ty1[None, :])
        rbx = jnp.minimum(bx2[i][:, None], tx2[None, :])
        rby = jnp.minimum(by2[i][:, None], ty2[None, :])
        iw = jnp.maximum(rbx - ltx, 0.0)
        ih = jnp.maximum(rby - lty, 0.0)
        inter = iw * ih
        union = a1[:, None] + a2[None, :] - inter
        iou_all = jnp.where(inter > 0, inter / union, 0.0)
        ious.append(jnp.max(iou_all * m[None, :], axis=1))
    iou = jnp.stack(ious, axis=0) * valid

    p = jnp.clip(val_ref[...], 1e-4, 1.0 - 1e-4)
    g_old = jnp.clip(gold_ref[...], 0.0, 1.0)
    g_new = jnp.clip(g_old + iou * 0.1, 0.0, 1.0)

    def terms(pp, gg):
        po = (gg == 1.0).astype(jnp.float32)
        omg = 1.0 - gg
        nw = omg * omg
        nw = nw * nw
        t_pos = jnp.log(pp) * (1.0 - pp) * (1.0 - pp) * po
        t_neg = jnp.log(1.0 - pp) * pp * pp * nw * (1.0 - po)
        return t_pos, t_neg, po

    pl_o, nl_o, po_o = terms(p, g_old)
    pl_n, nl_n, po_n = terms(p, g_new)
    pos_sum = pos_sum + jnp.sum((pl_n - pl_o) * valid)
    neg_sum = neg_sum + jnp.sum((nl_n - nl_o) * valid)
    num_pos = num_pos + jnp.sum((po_n - po_o) * valid)
    hm_loss = jnp.where(num_pos == 0.0, -neg_sum,
                        -(pos_sum + neg_sum) / jnp.maximum(num_pos, 1.0))

    rm = rm_ref[...]
    wh_sum = (jnp.sum(jnp.abs(pw0_ref[...] * rm - tw0_ref[...] * rm)) +
              jnp.sum(jnp.abs(pw1_ref[...] * rm - tw1_ref[...] * rm)))
    off_sum = (jnp.sum(jnp.abs(pr0_ref[...] * rm - tr0_ref[...] * rm)) +
               jnp.sum(jnp.abs(pr1_ref[...] * rm - tr1_ref[...] * rm)))
    msum = 2.0 * jnp.sum(rm)
    wh_loss = wh_sum / (msum + 1e-4)
    off_loss = off_sum / (msum + 1e-4)
    total = hm_loss + 0.1 * wh_loss + 1.0 * off_loss
    o_ref[...] = jnp.broadcast_to(total, (1, 1))


def _run_d(part, bx1, by1, bx2, by2, valid, val, gold, tb8,
           pw0, pw1, pr0, pr1, tw0, tw1, tr0, tr1, rm):
    return pl.pallas_call(
        _d_kernel,
        out_shape=jax.ShapeDtypeStruct((1, 1), jnp.float32),
    )(part, bx1, by1, bx2, by2, valid, val, gold, tb8,
      pw0, pw1, pr0, pr1, tw0, tw1, tr0, tr1, rm)


def kernel(hm, wh, reg, gt_hm, gt_wh, gt_reg, reg_mask, target_box, ind):
    cmax, carg, cg, w0a, w1a, r0a, r1a, part = _run_a(hm, gt_hm, wh, reg)
    thr = _run_s(cmax)

    cmax_i = cmax.reshape(B, NROW)
    carg_i = carg.reshape(B, NROW)
    cg_i = cg.reshape(B, NROW)
    bits = lax.bitcast_convert_type(cmax_i, jnp.int32)
    sel = bits >= thr[:, 0, 0][:, None]
    rank = jnp.cumsum(sel.astype(jnp.int32), axis=1) - 1
    slot = jnp.where(sel, rank, MAX_OBJS)
    rowids = jnp.broadcast_to(jnp.arange(NROW, dtype=jnp.int32)[None, :],
                              (B, NROW))
    rows = jnp.zeros((B, MAX_OBJS), jnp.int32).at[
        jnp.arange(B)[:, None], slot].set(
            rowids, mode="drop", indices_are_sorted=True, unique_indices=True)
    nsel = jnp.minimum(jnp.sum(sel, axis=1), MAX_OBJS)
    valid = (jnp.arange(MAX_OBJS)[None, :] < nsel[:, None]).astype(jnp.float32)

    take = jnp.take_along_axis
    cand = jnp.stack([cmax_i, carg.reshape(B, NROW), cg_i,
                      w0a.reshape(B, NROW), w1a.reshape(B, NROW),
                      r0a.reshape(B, NROW), r1a.reshape(B, NROW)], axis=1)
    dets = take(cand, rows[:, None, :], axis=2)
    val = dets[:, 0]
    xf = dets[:, 1]
    gold = dets[:, 2]
    w0 = dets[:, 3]
    w1 = dets[:, 4]
    r0 = dets[:, 5]
    r1 = dets[:, 6]
    y = rows % H
    xs = xf + r0
    ys = y.astype(jnp.float32) + r1
    bx1 = xs - w0 * 0.5
    by1 = ys - w1 * 0.5
    bx2 = xs + w0 * 0.5
    by2 = ys + w1 * 0.5

    tb8 = jnp.concatenate(
        [jnp.transpose(target_box), jnp.zeros((3, M), jnp.float32)], axis=0)

    indc = ind.astype(jnp.int32)
    whreg = jnp.concatenate([wh.reshape(B, 2, HW), reg.reshape(B, 2, HW)],
                            axis=1)
    pred4 = take(whreg, indc[:, None, :], axis=2)
    pw0 = pred4[:, 0]
    pw1 = pred4[:, 1]
    pr0 = pred4[:, 2]
    pr1 = pred4[:, 3]
    tw0 = gt_wh[:, :, 0]
    tw1 = gt_wh[:, :, 1]
    tr0 = gt_reg[:, :, 0]
    tr1 = gt_reg[:, :, 1]

    out = _run_d(part, bx1, by1, bx2, by2, valid, val, gold, tb8,
                 pw0, pw1, pr0, pr1, tw0, tw1, tr0, tr1, reg_mask)
    return out[0, 0]


# speedup vs baseline: 1.1049x; 1.1049x over previous
import functools

import jax
import jax.numpy as jnp
from jax import lax
from jax.experimental import pallas as pl
from jax.experimental.pallas import tpu as pltpu

B, C, H, W = 16, 80, 128, 128
K = 100
MAX_OBJS = 128
M = 512

PLANES = B * C
NROW = C * H
BLK = 16
HW = H * W


def _a_kernel(hm_ref, gt_ref, cmax_ref, carg_ref, cg_ref, part_ref):
    i = pl.program_id(0)
    x = hm_ref[...]
    g = gt_ref[...]
    s = jnp.clip(jax.nn.sigmoid(x), 1e-4, 1.0 - 1e-4)

    zc = jnp.zeros((BLK, H, 1), jnp.float32)
    left = jnp.concatenate([s[:, :, 1:], zc], axis=2)
    right = jnp.concatenate([zc, s[:, :, :-1]], axis=2)
    hx = jnp.maximum(jnp.maximum(left, right), s)
    zr = jnp.zeros((BLK, 1, W), jnp.float32)
    up = jnp.concatenate([hx[:, 1:, :], zr], axis=1)
    dn = jnp.concatenate([zr, hx[:, :-1, :]], axis=1)
    hmax = jnp.maximum(jnp.maximum(up, dn), hx)
    nm = jnp.where(hmax == s, s, 0.0)

    rmax = jnp.max(nm, axis=2)
    lane = lax.broadcasted_iota(jnp.int32, (BLK, H, W), 2)
    rarg = jnp.min(jnp.where(nm == rmax[:, :, None], lane, W), axis=2)
    onehot = lane == rarg[:, :, None]
    gsel = jnp.sum(jnp.where(onehot, g, 0.0), axis=2)
    cmax_ref[...] = rmax
    carg_ref[...] = rarg
    cg_ref[...] = gsel

    pos = (g == 1.0).astype(jnp.float32)
    one_m_g = 1.0 - g
    nw = one_m_g * one_m_g
    nw = nw * nw
    pos_loss = jnp.log(s) * (1.0 - s) * (1.0 - s) * pos
    neg_loss = jnp.log(1.0 - s) * s * s * nw * (1.0 - pos)

    def r(v):
        t = jnp.sum(v, axis=0)
        return jnp.sum(t.reshape(16, 8, 128), axis=0)

    part = jnp.stack([r(pos_loss), r(neg_loss), r(pos)], axis=0)

    @pl.when(i == 0)
    def _():
        part_ref[...] = jnp.zeros_like(part_ref)

    part_ref[...] += part


def _run_a(hm, gt_hm):
    return pl.pallas_call(
        _a_kernel,
        grid=(PLANES // BLK,),
        in_specs=[
            pl.BlockSpec((BLK, H, W), lambda i: (i, 0, 0)),
            pl.BlockSpec((BLK, H, W), lambda i: (i, 0, 0)),
        ],
        out_specs=[
            pl.BlockSpec((BLK, H), lambda i: (i, 0)),
            pl.BlockSpec((BLK, H), lambda i: (i, 0)),
            pl.BlockSpec((BLK, H), lambda i: (i, 0)),
            pl.BlockSpec((3, 8, 128), lambda i: (0, 0, 0)),
        ],
        out_shape=[
            jax.ShapeDtypeStruct((PLANES, H), jnp.float32),
            jax.ShapeDtypeStruct((PLANES, H), jnp.int32),
            jax.ShapeDtypeStruct((PLANES, H), jnp.float32),
            jax.ShapeDtypeStruct((3, 8, 128), jnp.float32),
        ],
    )(hm.reshape(PLANES, H, W), gt_hm.reshape(PLANES, H, W))


def _s_kernel(cm_ref, t_ref):
    v = cm_ref[...]
    bits = lax.bitcast_convert_type(v, jnp.int32)

    def body(_, lohi):
        lo, hi = lohi
        mid = (lo + hi) // 2
        cnt = jnp.sum((bits >= mid).astype(jnp.int32))
        take = cnt >= K
        return jnp.where(take, mid, lo), jnp.where(take, hi, mid)

    lo, _ = lax.fori_loop(0, 31, body, (jnp.int32(0), jnp.int32(0x3F800001)))
    t_ref[...] = jnp.full((1, 8, 128), lo, jnp.int32)


def _run_s(cmax):
    return pl.pallas_call(
        _s_kernel,
        grid=(B,),
        in_specs=[pl.BlockSpec((1, C, H), lambda i: (i, 0, 0))],
        out_specs=pl.BlockSpec((1, 8, 128), lambda i: (i, 0, 0)),
        out_shape=jax.ShapeDtypeStruct((B, 8, 128), jnp.int32),
    )(cmax.reshape(B, C, H))


def _d_kernel(part_ref, bx1_ref, by1_ref, bx2_ref, by2_ref, valid_ref,
              val_ref, gold_ref, tb_ref,
              pw0_ref, pw1_ref, pr0_ref, pr1_ref,
              tw0_ref, tw1_ref, tr0_ref, tr1_ref, rm_ref, o_ref):
    part = part_ref[...]
    pos_sum = jnp.sum(part[0])
    neg_sum = jnp.sum(part[1])
    num_pos = jnp.sum(part[2])

    tb = tb_ref[...]
    tx1 = tb[0]
    ty1 = tb[1]
    tx2 = tb[2]
    ty2 = tb[3]
    tbi = tb[4]
    a2 = (tx2 - tx1) * (ty2 - ty1)

    bx1 = bx1_ref[...]
    by1 = by1_ref[...]
    bx2 = bx2_ref[...]
    by2 = by2_ref[...]
    valid = valid_ref[...]

    ious = []
    for i in range(B):
        m = (tbi == float(i)).astype(jnp.float32)
        a1 = (bx2[i] - bx1[i]) * (by2[i] - by1[i])
        ltx = jnp.maximum(bx1[i][:, None], tx1[None, :])
        lty = jnp.maximum(by1[i][:, None], ty1[None, :])
        rbx = jnp.minimum(bx2[i][:, None], tx2[None, :])
        rby = jnp.minimum(by2[i][:, None], ty2[None, :])
        iw = jnp.maximum(rbx - ltx, 0.0)
        ih = jnp.maximum(rby - lty, 0.0)
        inter = iw * ih
        union = a1[:, None] + a2[None, :] - inter
        iou_all = jnp.where(inter > 0, inter / union, 0.0)
        ious.append(jnp.max(iou_all * m[None, :], axis=1))
    iou = jnp.stack(ious, axis=0) * valid

    p = jnp.clip(val_ref[...], 1e-4, 1.0 - 1e-4)
    g_old = jnp.clip(gold_ref[...], 0.0, 1.0)
    g_new = jnp.clip(g_old + iou * 0.1, 0.0, 1.0)

    def terms(pp, gg):
        po = (gg == 1.0).astype(jnp.float32)
        omg = 1.0 - gg
        nw = omg * omg
        nw = nw * nw
        t_pos = jnp.log(pp) * (1.0 - pp) * (1.0 - pp) * po
        t_neg = jnp.log(1.0 - pp) * pp * pp * nw * (1.0 - po)
        return t_pos, t_neg, po

    pl_o, nl_o, po_o = terms(p, g_old)
    pl_n, nl_n, po_n = terms(p, g_new)
    pos_sum = pos_sum + jnp.sum((pl_n - pl_o) * valid)
    neg_sum = neg_sum + jnp.sum((nl_n - nl_o) * valid)
    num_pos = num_pos + jnp.sum((po_n - po_o) * valid)
    hm_loss = jnp.where(num_pos == 0.0, -neg_sum,
                        -(pos_sum + neg_sum) / jnp.maximum(num_pos, 1.0))

    rm = rm_ref[...]
    wh_sum = (jnp.sum(jnp.abs(pw0_ref[...] * rm - tw0_ref[...] * rm)) +
              jnp.sum(jnp.abs(pw1_ref[...] * rm - tw1_ref[...] * rm)))
    off_sum = (jnp.sum(jnp.abs(pr0_ref[...] * rm - tr0_ref[...] * rm)) +
               jnp.sum(jnp.abs(pr1_ref[...] * rm - tr1_ref[...] * rm)))
    msum = 2.0 * jnp.sum(rm)
    wh_loss = wh_sum / (msum + 1e-4)
    off_loss = off_sum / (msum + 1e-4)
    total = hm_loss + 0.1 * wh_loss + 1.0 * off_loss
    o_ref[...] = jnp.broadcast_to(total, (1, 1))


def _run_d(part, bx1, by1, bx2, by2, valid, val, gold, tb8,
           pw0, pw1, pr0, pr1, tw0, tw1, tr0, tr1, rm):
    return pl.pallas_call(
        _d_kernel,
        out_shape=jax.ShapeDtypeStruct((1, 1), jnp.float32),
    )(part, bx1, by1, bx2, by2, valid, val, gold, tb8,
      pw0, pw1, pr0, pr1, tw0, tw1, tr0, tr1, rm)


def kernel(hm, wh, reg, gt_hm, gt_wh, gt_reg, reg_mask, target_box, ind):
    cmax, carg, cg, part = _run_a(hm, gt_hm)
    thr = _run_s(cmax)

    cmax_i = cmax.reshape(B, NROW)
    carg_i = carg.reshape(B, NROW)
    cg_i = cg.reshape(B, NROW)
    bits = lax.bitcast_convert_type(cmax_i, jnp.int32)
    sel = bits >= thr[:, 0, 0][:, None]
    rank = jnp.cumsum(sel.astype(jnp.int32), axis=1) - 1
    slot = jnp.where(sel, rank, MAX_OBJS)
    rowids = jnp.broadcast_to(jnp.arange(NROW, dtype=jnp.int32)[None, :],
                              (B, NROW))
    rows = jnp.zeros((B, MAX_OBJS), jnp.int32).at[
        jnp.arange(B)[:, None], slot].set(
            rowids, mode="drop", indices_are_sorted=True, unique_indices=True)
    nsel = jnp.minimum(jnp.sum(sel, axis=1), MAX_OBJS)
    valid = (jnp.arange(MAX_OBJS)[None, :] < nsel[:, None]).astype(jnp.float32)

    take = jnp.take_along_axis
    cand = jnp.stack([cmax_i, cg_i], axis=1)
    dets2 = take(cand, rows[:, None, :], axis=2)
    val = dets2[:, 0]
    gold = dets2[:, 1]
    argx = take(carg_i, rows, axis=1)
    y = rows % H
    x = argx
    flat = y * W + x
    whreg = jnp.concatenate([wh.reshape(B, 2, HW), reg.reshape(B, 2, HW)],
                            axis=1)
    det4 = take(whreg, flat[:, None, :], axis=2)
    w0 = det4[:, 0]
    w1 = det4[:, 1]
    r0 = det4[:, 2]
    r1 = det4[:, 3]
    xs = x.astype(jnp.float32) + r0
    ys = y.astype(jnp.float32) + r1
    bx1 = xs - w0 * 0.5
    by1 = ys - w1 * 0.5
    bx2 = xs + w0 * 0.5
    by2 = ys + w1 * 0.5

    tb8 = jnp.concatenate(
        [jnp.transpose(target_box), jnp.zeros((3, M), jnp.float32)], axis=0)

    indc = ind.astype(jnp.int32)
    pred4 = take(whreg, indc[:, None, :], axis=2)
    pw0 = pred4[:, 0]
    pw1 = pred4[:, 1]
    pr0 = pred4[:, 2]
    pr1 = pred4[:, 3]
    tw0 = gt_wh[:, :, 0]
    tw1 = gt_wh[:, :, 1]
    tr0 = gt_reg[:, :, 0]
    tr1 = gt_reg[:, :, 1]

    out = _run_d(part, bx1, by1, bx2, by2, valid, val, gold, tb8,
                 pw0, pw1, pr0, pr1, tw0, tw1, tr0, tr1, reg_mask)
    return out[0, 0]


# speedup vs baseline: 1.1694x; 1.0583x over previous
import functools

import jax
import jax.numpy as jnp
from jax import lax
from jax.experimental import pallas as pl
from jax.experimental.pallas import tpu as pltpu

B, C, H, W = 16, 80, 128, 128
K = 100
MAX_OBJS = 128
M = 512

PLANES = B * C
NROW = C * H
BLK = 16
HW = H * W


def _a_kernel(hm_ref, gt_ref, cmax_ref, carg_ref, cg_ref, part_ref):
    i = pl.program_id(0)
    x = hm_ref[...]
    g = gt_ref[...]
    s = jnp.clip(jax.nn.sigmoid(x), 1e-4, 1.0 - 1e-4)

    zc = jnp.zeros((BLK, H, 1), jnp.float32)
    left = jnp.concatenate([s[:, :, 1:], zc], axis=2)
    right = jnp.concatenate([zc, s[:, :, :-1]], axis=2)
    hx = jnp.maximum(jnp.maximum(left, right), s)
    zr = jnp.zeros((BLK, 1, W), jnp.float32)
    up = jnp.concatenate([hx[:, 1:, :], zr], axis=1)
    dn = jnp.concatenate([zr, hx[:, :-1, :]], axis=1)
    hmax = jnp.maximum(jnp.maximum(up, dn), hx)
    nm = jnp.where(hmax == s, s, 0.0)

    rmax = jnp.max(nm, axis=2)
    lane = lax.broadcasted_iota(jnp.int32, (BLK, H, W), 2)
    rarg = jnp.min(jnp.where(nm == rmax[:, :, None], lane, W), axis=2)
    onehot = lane == rarg[:, :, None]
    gsel = jnp.sum(jnp.where(onehot, g, 0.0), axis=2)
    cmax_ref[...] = rmax
    carg_ref[...] = rarg
    cg_ref[...] = gsel

    pos = (g == 1.0).astype(jnp.float32)
    one_m_g = 1.0 - g
    nw = one_m_g * one_m_g
    nw = nw * nw
    pos_loss = jnp.log(s) * (1.0 - s) * (1.0 - s) * pos
    neg_loss = jnp.log(1.0 - s) * s * s * nw * (1.0 - pos)

    def r(v):
        t = jnp.sum(v, axis=0)
        return jnp.sum(t.reshape(16, 8, 128), axis=0)

    part = jnp.stack([r(pos_loss), r(neg_loss), r(pos)], axis=0)

    @pl.when(i == 0)
    def _():
        part_ref[...] = jnp.zeros_like(part_ref)

    part_ref[...] += part


def _run_a(hm, gt_hm):
    return pl.pallas_call(
        _a_kernel,
        grid=(PLANES // BLK,),
        in_specs=[
            pl.BlockSpec((BLK, H, W), lambda i: (i, 0, 0)),
            pl.BlockSpec((BLK, H, W), lambda i: (i, 0, 0)),
        ],
        out_specs=[
            pl.BlockSpec((BLK, H), lambda i: (i, 0)),
            pl.BlockSpec((BLK, H), lambda i: (i, 0)),
            pl.BlockSpec((BLK, H), lambda i: (i, 0)),
            pl.BlockSpec((3, 8, 128), lambda i: (0, 0, 0)),
        ],
        out_shape=[
            jax.ShapeDtypeStruct((PLANES, H), jnp.float32),
            jax.ShapeDtypeStruct((PLANES, H), jnp.int32),
            jax.ShapeDtypeStruct((PLANES, H), jnp.float32),
            jax.ShapeDtypeStruct((3, 8, 128), jnp.float32),
        ],
    )(hm.reshape(PLANES, H, W), gt_hm.reshape(PLANES, H, W))


def _s_kernel(cm_ref, t_ref):
    v = cm_ref[...]
    bits = lax.bitcast_convert_type(v, jnp.int32)

    def body(_, lohi):
        lo, hi = lohi
        mid = (lo + hi) // 2
        cnt = jnp.sum((bits >= mid).astype(jnp.int32))
        take = cnt >= K
        return jnp.where(take, mid, lo), jnp.where(take, hi, mid)

    lo, _ = lax.fori_loop(0, 31, body, (jnp.int32(0), jnp.int32(0x3F800001)))
    t_ref[...] = jnp.full((1, 8, 128), lo, jnp.int32)


def _run_s(cmax):
    return pl.pallas_call(
        _s_kernel,
        grid=(B,),
        in_specs=[pl.BlockSpec((1, C, H), lambda i: (i, 0, 0))],
        out_specs=pl.BlockSpec((1, 8, 128), lambda i: (i, 0, 0)),
        out_shape=jax.ShapeDtypeStruct((B, 8, 128), jnp.int32),
    )(cmax.reshape(B, C, H))


def _d_kernel(part_ref, bx1_ref, by1_ref, bx2_ref, by2_ref, valid_ref,
              val_ref, gold_ref, tb_ref,
              pw0_ref, pw1_ref, pr0_ref, pr1_ref,
              tw0_ref, tw1_ref, tr0_ref, tr1_ref, rm_ref, o_ref):
    part = part_ref[...]
    pos_sum = jnp.sum(part[0])
    neg_sum = jnp.sum(part[1])
    num_pos = jnp.sum(part[2])

    tb = tb_ref[...]
    tx1 = tb[0]
    ty1 = tb[1]
    tx2 = tb[2]
    ty2 = tb[3]
    tbi = tb[4]
    a2 = (tx2 - tx1) * (ty2 - ty1)

    bx1 = bx1_ref[...]
    by1 = by1_ref[...]
    bx2 = bx2_ref[...]
    by2 = by2_ref[...]
    valid = valid_ref[...]

    ious = []
    for i in range(B):
        m = (tbi == float(i)).astype(jnp.float32)
        a1 = (bx2[i] - bx1[i]) * (by2[i] - by1[i])
        ltx = jnp.maximum(bx1[i][:, None], tx1[None, :])
        lty = jnp.maximum(by1[i][:, None], ty1[None, :])
        rbx = jnp.minimum(bx2[i][:, None], tx2[None, :])
        rby = jnp.minimum(by2[i][:, None], ty2[None, :])
        iw = jnp.maximum(rbx - ltx, 0.0)
        ih = jnp.maximum(rby - lty, 0.0)
        inter = iw * ih
        union = a1[:, None] + a2[None, :] - inter
        iou_all = jnp.where(inter > 0, inter / union, 0.0)
        ious.append(jnp.max(iou_all * m[None, :], axis=1))
    iou = jnp.stack(ious, axis=0) * valid

    p = jnp.clip(val_ref[...], 1e-4, 1.0 - 1e-4)
    g_old = jnp.clip(gold_ref[...], 0.0, 1.0)
    g_new = jnp.clip(g_old + iou * 0.1, 0.0, 1.0)

    def terms(pp, gg):
        po = (gg == 1.0).astype(jnp.float32)
        omg = 1.0 - gg
        nw = omg * omg
        nw = nw * nw
        t_pos = jnp.log(pp) * (1.0 - pp) * (1.0 - pp) * po
        t_neg = jnp.log(1.0 - pp) * pp * pp * nw * (1.0 - po)
        return t_pos, t_neg, po

    pl_o, nl_o, po_o = terms(p, g_old)
    pl_n, nl_n, po_n = terms(p, g_new)
    pos_sum = pos_sum + jnp.sum((pl_n - pl_o) * valid)
    neg_sum = neg_sum + jnp.sum((nl_n - nl_o) * valid)
    num_pos = num_pos + jnp.sum((po_n - po_o) * valid)
    hm_loss = jnp.where(num_pos == 0.0, -neg_sum,
                        -(pos_sum + neg_sum) / jnp.maximum(num_pos, 1.0))

    rm = rm_ref[...]
    wh_sum = (jnp.sum(jnp.abs(pw0_ref[...] * rm - tw0_ref[...] * rm)) +
              jnp.sum(jnp.abs(pw1_ref[...] * rm - tw1_ref[...] * rm)))
    off_sum = (jnp.sum(jnp.abs(pr0_ref[...] * rm - tr0_ref[...] * rm)) +
               jnp.sum(jnp.abs(pr1_ref[...] * rm - tr1_ref[...] * rm)))
    msum = 2.0 * jnp.sum(rm)
    wh_loss = wh_sum / (msum + 1e-4)
    off_loss = off_sum / (msum + 1e-4)
    total = hm_loss + 0.1 * wh_loss + 1.0 * off_loss
    o_ref[...] = jnp.broadcast_to(total, (1, 1))


def _run_d(part, bx1, by1, bx2, by2, valid, val, gold, tb8,
           pw0, pw1, pr0, pr1, tw0, tw1, tr0, tr1, rm):
    return pl.pallas_call(
        _d_kernel,
        out_shape=jax.ShapeDtypeStruct((1, 1), jnp.float32),
    )(part, bx1, by1, bx2, by2, valid, val, gold, tb8,
      pw0, pw1, pr0, pr1, tw0, tw1, tr0, tr1, rm)


def kernel(hm, wh, reg, gt_hm, gt_wh, gt_reg, reg_mask, target_box, ind):
    cmax, carg, cg, part = _run_a(hm, gt_hm)
    thr = _run_s(cmax)

    cmax_i = cmax.reshape(B, NROW)
    carg_i = carg.reshape(B, NROW)
    cg_i = cg.reshape(B, NROW)
    bits = lax.bitcast_convert_type(cmax_i, jnp.int32)
    sel = bits >= thr[:, 0, 0][:, None]
    rank = jnp.cumsum(sel.astype(jnp.int32), axis=1) - 1
    slot = jnp.where(sel, rank, MAX_OBJS)
    rowids = jnp.broadcast_to(jnp.arange(NROW, dtype=jnp.int32)[None, :],
                              (B, NROW))
    rows = jnp.zeros((B, MAX_OBJS), jnp.int32).at[
        jnp.arange(B)[:, None], slot].set(rowids, mode="drop")
    nsel = jnp.minimum(jnp.sum(sel, axis=1), MAX_OBJS)
    valid = (jnp.arange(MAX_OBJS)[None, :] < nsel[:, None]).astype(jnp.float32)

    take = jnp.take_along_axis
    val = take(cmax_i, rows, axis=1)
    argx = take(carg_i, rows, axis=1)
    gold = take(cg_i, rows, axis=1)
    y = rows % H
    x = argx
    flat = y * W + x
    wh_f = wh.reshape(B, 2, HW)
    reg_f = reg.reshape(B, 2, HW)
    w0 = take(wh_f[:, 0], flat, axis=1)
    w1 = take(wh_f[:, 1], flat, axis=1)
    r0 = take(reg_f[:, 0], flat, axis=1)
    r1 = take(reg_f[:, 1], flat, axis=1)
    xs = x.astype(jnp.float32) + r0
    ys = y.astype(jnp.float32) + r1
    bx1 = xs - w0 * 0.5
    by1 = ys - w1 * 0.5
    bx2 = xs + w0 * 0.5
    by2 = ys + w1 * 0.5

    tb8 = jnp.concatenate(
        [jnp.transpose(target_box), jnp.zeros((3, M), jnp.float32)], axis=0)

    indc = ind.astype(jnp.int32)
    pw0 = take(wh_f[:, 0], indc, axis=1)
    pw1 = take(wh_f[:, 1], indc, axis=1)
    pr0 = take(reg_f[:, 0], indc, axis=1)
    pr1 = take(reg_f[:, 1], indc, axis=1)
    tw0 = gt_wh[:, :, 0]
    tw1 = gt_wh[:, :, 1]
    tr0 = gt_reg[:, :, 0]
    tr1 = gt_reg[:, :, 1]

    out = _run_d(part, bx1, by1, bx2, by2, valid, val, gold, tb8,
                 pw0, pw1, pr0, pr1, tw0, tw1, tr0, tr1, reg_mask)
    return out[0, 0]
